# baseline (speedup 1.0000x reference)
import jax
import jax.numpy as jnp
from jax.experimental import pallas as pl
from jax.experimental.pallas import tpu as pltpu

_MIB = 1024 * 1024


def _matmul_kernel(x_ref, w_ref, o_ref):
    o_ref[...] = jax.lax.dot_general(
        x_ref[...],
        w_ref[...],
        dimension_numbers=(((1,), (1,)), ((), ())),
        preferred_element_type=jnp.float32,
    )


def kernel(x, weight):
    M, K = x.shape
    N = weight.shape[0]
    out_dtype = x.dtype

    x_bf = x.astype(jnp.bfloat16)
    w_nk = weight.astype(jnp.bfloat16)

    tm = 512
    grid_m = M // tm

    footprint = K * N * 2 + 2 * tm * K * 2 + 2 * tm * N * 4

    return pl.pallas_call(
        _matmul_kernel,
        out_shape=jax.ShapeDtypeStruct((M, N), out_dtype),
        grid=(grid_m,),
        in_specs=[
            pl.BlockSpec((tm, K), lambda i: (i, 0)),
            pl.BlockSpec((N, K), lambda i: (0, 0)),
        ],
        out_specs=pl.BlockSpec((tm, N), lambda i: (i, 0)),
        compiler_params=pltpu.CompilerParams(
            dimension_semantics=("parallel",),
            vmem_limit_bytes=int(footprint + 8 * _MIB),
        ),
        cost_estimate=pl.CostEstimate(
            flops=2 * M * N * K,
            transcendentals=0,
            bytes_accessed=M * K * 2 + K * N * 2 + M * N * 4,
        ),
    )(x_bf, w_nk)


# speedup vs baseline: 1.3938x; 1.3938x over previous
import jax
import jax.numpy as jnp
from jax.experimental import pallas as pl
from jax.experimental.pallas import tpu as pltpu

_MIB = 1024 * 1024


def _matmul_kernel(x_ref, w_ref, o_ref):
    o_ref[...] = jax.lax.dot_general(
        x_ref[...].astype(jnp.bfloat16),
        w_ref[...],
        dimension_numbers=(((1,), (1,)), ((), ())),
        preferred_element_type=jnp.float32,
    )


def kernel(x, weight):
    M, K = x.shape
    N = weight.shape[0]
    out_dtype = x.dtype

    w_nk = weight.astype(jnp.bfloat16)

    bm, bn = 1024, 1024
    grid = (M // bm, N // bn)

    footprint = 2 * bm * K * 4 + 2 * bn * K * 2 + 2 * bm * bn * 4

    return pl.pallas_call(
        _matmul_kernel,
        out_shape=jax.ShapeDtypeStruct((M, N), out_dtype),
        grid=grid,
        in_specs=[
            pl.BlockSpec((bm, K), lambda i, j: (i, 0)),
            pl.BlockSpec((bn, K), lambda i, j: (j, 0)),
        ],
        out_specs=pl.BlockSpec((bm, bn), lambda i, j: (i, j)),
        compiler_params=pltpu.CompilerParams(
            dimension_semantics=("parallel", "parallel"),
            vmem_limit_bytes=int(footprint + 8 * _MIB),
        ),
        cost_estimate=pl.CostEstimate(
            flops=2 * M * N * K,
            transcendentals=0,
            bytes_accessed=M * K * 4 + K * N * 2 + M * N * 4,
        ),
    )(x, w_nk)


# speedup vs baseline: 1.4663x; 1.0520x over previous
import jax
import jax.numpy as jnp
from jax.experimental import pallas as pl
from jax.experimental.pallas import tpu as pltpu

_MIB = 1024 * 1024


def _matmul_kernel(x_ref, w_ref, o_ref, w_bf_ref):
    @pl.when(pl.program_id(0) == 0)
    def _():
        w_bf_ref[...] = w_ref[...].T.astype(jnp.bfloat16)

    o_ref[...] = jnp.dot(
        x_ref[...].astype(jnp.bfloat16),
        w_bf_ref[...],
        preferred_element_type=jnp.float32,
    )


def kernel(x, weight):
    M, K = x.shape
    N = weight.shape[0]
    out_dtype = x.dtype

    tm = 512
    grid_m = M // tm

    footprint = K * N * 4 + K * N * 2 + 2 * tm * (K + N) * 4

    return pl.pallas_call(
        _matmul_kernel,
        out_shape=jax.ShapeDtypeStruct((M, N), out_dtype),
        grid=(grid_m,),
        in_specs=[
            pl.BlockSpec((tm, K), lambda i: (i, 0)),
            pl.BlockSpec((N, K), lambda i: (0, 0)),
        ],
        out_specs=pl.BlockSpec((tm, N), lambda i: (i, 0)),
        scratch_shapes=[pltpu.VMEM((K, N), jnp.bfloat16)],
        compiler_params=pltpu.CompilerParams(
            dimension_semantics=("arbitrary",),
            vmem_limit_bytes=int(footprint + 8 * _MIB),
        ),
        cost_estimate=pl.CostEstimate(
            flops=2 * M * N * K,
            transcendentals=0,
            bytes_accessed=M * K * 4 + K * N * 4 + M * N * 4,
        ),
    )(x, weight)
